# baseline (speedup 1.0000x reference)
import functools

import jax
import jax.numpy as jnp
from jax import lax
from jax.experimental import pallas as pl
from jax.experimental.pallas import tpu as pltpu
from jax.experimental.pallas import tpu_sc as plsc

_NC = 2
_NS = 16
_NW = _NC * _NS


_K = 16


def _gather_sc(user_indices, item_indices, user_table, item_table):
    B = user_indices.shape[0]
    D = user_table.shape[1]
    b_per_w = B // _NW
    mesh = plsc.VectorSubcoreMesh(core_axis_name="c", subcore_axis_name="s")

    @functools.partial(
        pl.kernel,
        mesh=mesh,
        out_type=(
            jax.ShapeDtypeStruct((B, D), jnp.float32),
            jax.ShapeDtypeStruct((B, D), jnp.float32),
        ),
        scratch_types=[
            pltpu.VMEM((b_per_w,), jnp.int32),
            pltpu.VMEM((b_per_w,), jnp.int32),
            pltpu.SemaphoreType.DMA,
            pltpu.SemaphoreType.DMA,
        ],
    )
    def gather(uidx_hbm, iidx_hbm, utab_hbm, itab_hbm, uout_hbm, iout_hbm,
               uidx_v, iidx_v, usem, isem):
        wid = lax.axis_index("s") * _NC + lax.axis_index("c")
        base = wid * b_per_w
        pltpu.sync_copy(uidx_hbm.at[pl.ds(base, b_per_w)], uidx_v)
        pltpu.sync_copy(iidx_hbm.at[pl.ds(base, b_per_w)], iidx_v)

        def issue(c, carry):
            uvec = uidx_v[pl.ds(c * _K, _K)]
            ivec = iidx_v[pl.ds(c * _K, _K)]
            for j in range(_K):
                i = base + c * _K + j
                pltpu.async_copy(utab_hbm.at[uvec[j]], uout_hbm.at[i], usem)
                pltpu.async_copy(itab_hbm.at[ivec[j]], iout_hbm.at[i], isem)
            return carry

        lax.fori_loop(0, b_per_w // _K, issue, 0)
        pltpu.make_async_copy(utab_hbm.at[pl.ds(0, b_per_w)],
                              uout_hbm.at[pl.ds(0, b_per_w)], usem).wait()
        pltpu.make_async_copy(itab_hbm.at[pl.ds(0, b_per_w)],
                              iout_hbm.at[pl.ds(0, b_per_w)], isem).wait()

    return gather(user_indices, item_indices, user_table, item_table)


def _mlp_body(ue_ref, ie_ref, w1u_ref, w1i_ref, b1_ref, w2_ref, b2_ref, out_ref):
    h = jnp.dot(ue_ref[...], w1u_ref[...], preferred_element_type=jnp.float32)
    h = h + jnp.dot(ie_ref[...], w1i_ref[...], preferred_element_type=jnp.float32)
    h = jnp.maximum(h + b1_ref[...], 0.0)
    out_ref[...] = jnp.dot(h, w2_ref[...], preferred_element_type=jnp.float32) + b2_ref[...]


def _mlp_tc(ue, ie, W1u, W1i, b1, W2, b2, block_b=2048):
    B, D = ue.shape
    H = W1u.shape[1]
    grid = (B // block_b,)
    return pl.pallas_call(
        _mlp_body,
        grid=grid,
        in_specs=[
            pl.BlockSpec((block_b, D), lambda i: (i, 0)),
            pl.BlockSpec((block_b, D), lambda i: (i, 0)),
            pl.BlockSpec((D, H), lambda i: (0, 0)),
            pl.BlockSpec((D, H), lambda i: (0, 0)),
            pl.BlockSpec((1, H), lambda i: (0, 0)),
            pl.BlockSpec((H, 1), lambda i: (0, 0)),
            pl.BlockSpec((1, 1), lambda i: (0, 0)),
        ],
        out_specs=pl.BlockSpec((block_b, 1), lambda i: (i, 0)),
        out_shape=jax.ShapeDtypeStruct((B, 1), jnp.float32),
    )(ue, ie, W1u, W1i, b1, W2, b2)


def kernel(user_indices, item_indices, user_table, item_table, W1, b1, W2, b2):
    D = user_table.shape[1]
    ue, ie = _gather_sc(
        user_indices.astype(jnp.int32),
        item_indices.astype(jnp.int32),
        user_table,
        item_table,
    )
    return _mlp_tc(
        ue, ie,
        W1[:D], W1[D:],
        b1.reshape(1, -1), W2, b2.reshape(1, 1),
    )


# speedup vs baseline: 1.0001x; 1.0001x over previous
import functools

import jax
import jax.numpy as jnp
from jax import lax
from jax.experimental import pallas as pl
from jax.experimental.pallas import tpu as pltpu
from jax.experimental.pallas import tpu_sc as plsc

_NC = 2
_NS = 16
_NW = _NC * _NS


_K = 16


def _gather_sc(user_indices, item_indices, user_table, item_table):
    B = user_indices.shape[0]
    D = user_table.shape[1]
    b_per_w = B // _NW
    mesh = plsc.VectorSubcoreMesh(core_axis_name="c", subcore_axis_name="s")

    @functools.partial(
        pl.kernel,
        mesh=mesh,
        out_type=(
            jax.ShapeDtypeStruct((B, D), jnp.float32),
            jax.ShapeDtypeStruct((B, D), jnp.float32),
        ),
        scratch_types=[
            pltpu.VMEM((b_per_w,), jnp.int32),
            pltpu.VMEM((b_per_w,), jnp.int32),
            pltpu.SemaphoreType.DMA,
            pltpu.SemaphoreType.DMA,
        ],
        compiler_params=pltpu.CompilerParams(needs_layout_passes=False),
    )
    def gather(uidx_hbm, iidx_hbm, utab_hbm, itab_hbm, uout_hbm, iout_hbm,
               uidx_v, iidx_v, usem, isem):
        wid = lax.axis_index("s") * _NC + lax.axis_index("c")
        base = wid * b_per_w
        pltpu.sync_copy(uidx_hbm.at[pl.ds(base, b_per_w)], uidx_v)
        pltpu.sync_copy(iidx_hbm.at[pl.ds(base, b_per_w)], iidx_v)

        def issue(c, carry):
            uvec = uidx_v[pl.ds(c * _K, _K)]
            ivec = iidx_v[pl.ds(c * _K, _K)]
            for j in range(_K):
                i = base + c * _K + j
                pltpu.async_copy(utab_hbm.at[uvec[j]], uout_hbm.at[i], usem)
                pltpu.async_copy(itab_hbm.at[ivec[j]], iout_hbm.at[i], isem)
            return carry

        lax.fori_loop(0, b_per_w // _K, issue, 0)
        pltpu.make_async_copy(utab_hbm.at[pl.ds(0, b_per_w)],
                              uout_hbm.at[pl.ds(0, b_per_w)], usem).wait()
        pltpu.make_async_copy(itab_hbm.at[pl.ds(0, b_per_w)],
                              iout_hbm.at[pl.ds(0, b_per_w)], isem).wait()

    return gather(user_indices, item_indices, user_table, item_table)


def _mlp_body(ue_ref, ie_ref, w1u_ref, w1i_ref, b1_ref, w2_ref, b2_ref, out_ref):
    h = jnp.dot(ue_ref[...], w1u_ref[...], preferred_element_type=jnp.float32)
    h = h + jnp.dot(ie_ref[...], w1i_ref[...], preferred_element_type=jnp.float32)
    h = jnp.maximum(h + b1_ref[...], 0.0)
    out_ref[...] = jnp.dot(h, w2_ref[...], preferred_element_type=jnp.float32) + b2_ref[...]


def _mlp_tc(ue, ie, W1u, W1i, b1, W2, b2, block_b=2048):
    B, D = ue.shape
    H = W1u.shape[1]
    grid = (B // block_b,)
    return pl.pallas_call(
        _mlp_body,
        grid=grid,
        in_specs=[
            pl.BlockSpec((block_b, D), lambda i: (i, 0)),
            pl.BlockSpec((block_b, D), lambda i: (i, 0)),
            pl.BlockSpec((D, H), lambda i: (0, 0)),
            pl.BlockSpec((D, H), lambda i: (0, 0)),
            pl.BlockSpec((1, H), lambda i: (0, 0)),
            pl.BlockSpec((H, 1), lambda i: (0, 0)),
            pl.BlockSpec((1, 1), lambda i: (0, 0)),
        ],
        out_specs=pl.BlockSpec((block_b, 1), lambda i: (i, 0)),
        out_shape=jax.ShapeDtypeStruct((B, 1), jnp.float32),
    )(ue, ie, W1u, W1i, b1, W2, b2)


def kernel(user_indices, item_indices, user_table, item_table, W1, b1, W2, b2):
    D = user_table.shape[1]
    ue, ie = _gather_sc(
        user_indices.astype(jnp.int32),
        item_indices.astype(jnp.int32),
        user_table,
        item_table,
    )
    return _mlp_tc(
        ue, ie,
        W1[:D], W1[D:],
        b1.reshape(1, -1), W2, b2.reshape(1, 1),
    )


# speedup vs baseline: 2.0553x; 2.0551x over previous
import functools

import jax
import jax.numpy as jnp
from jax import lax
from jax.experimental import pallas as pl
from jax.experimental.pallas import tpu as pltpu
from jax.experimental.pallas import tpu_sc as plsc

_NC = 2
_NS = 16
_NW = _NC * _NS


_K = 16


def _gather_sc(user_indices, item_indices, user_table, item_table):
    B = user_indices.shape[0]
    D = user_table.shape[1]
    b_per_w = B // _NW
    half = b_per_w // 2
    mesh = plsc.VectorSubcoreMesh(core_axis_name="c", subcore_axis_name="s")

    @functools.partial(
        pl.kernel,
        mesh=mesh,
        out_type=(
            jax.ShapeDtypeStruct((B, D), jnp.float32),
            jax.ShapeDtypeStruct((B, D), jnp.float32),
        ),
        scratch_types=[
            pltpu.VMEM((b_per_w,), jnp.int32),
            pltpu.VMEM((b_per_w,), jnp.int32),
            pltpu.VMEM((half, D), jnp.float32),
            pltpu.VMEM((half, D), jnp.float32),
            pltpu.SemaphoreType.DMA,
            pltpu.SemaphoreType.DMA,
        ],
    )
    def gather(uidx_hbm, iidx_hbm, utab_hbm, itab_hbm, uout_hbm, iout_hbm,
               uidx_v, iidx_v, ubuf, ibuf, usem, isem):
        wid = lax.axis_index("s") * _NC + lax.axis_index("c")
        base = wid * b_per_w
        pltpu.sync_copy(uidx_hbm.at[pl.ds(base, b_per_w)], uidx_v)
        pltpu.sync_copy(iidx_hbm.at[pl.ds(base, b_per_w)], iidx_v)

        for r in range(2):
            def issue(c, carry):
                uvec = uidx_v[pl.ds(r * half + c * _K, _K)]
                ivec = iidx_v[pl.ds(r * half + c * _K, _K)]
                for j in range(_K):
                    i = c * _K + j
                    pltpu.async_copy(utab_hbm.at[uvec[j]], ubuf.at[i], usem)
                    pltpu.async_copy(itab_hbm.at[ivec[j]], ibuf.at[i], isem)
                return carry

            lax.fori_loop(0, half // _K, issue, 0)
            pltpu.make_async_copy(utab_hbm.at[pl.ds(0, half)], ubuf, usem).wait()
            pltpu.sync_copy(ubuf, uout_hbm.at[pl.ds(base + r * half, half)])
            pltpu.make_async_copy(itab_hbm.at[pl.ds(0, half)], ibuf, isem).wait()
            pltpu.sync_copy(ibuf, iout_hbm.at[pl.ds(base + r * half, half)])

    return gather(user_indices, item_indices, user_table, item_table)


def _mlp_body(ue_ref, ie_ref, w1u_ref, w1i_ref, b1_ref, w2_ref, b2_ref, out_ref):
    h = jnp.dot(ue_ref[...], w1u_ref[...], preferred_element_type=jnp.float32)
    h = h + jnp.dot(ie_ref[...], w1i_ref[...], preferred_element_type=jnp.float32)
    h = jnp.maximum(h + b1_ref[...], 0.0)
    out_ref[...] = jnp.dot(h, w2_ref[...], preferred_element_type=jnp.float32) + b2_ref[...]


def _mlp_tc(ue, ie, W1u, W1i, b1, W2, b2, block_b=2048):
    B, D = ue.shape
    H = W1u.shape[1]
    grid = (B // block_b,)
    return pl.pallas_call(
        _mlp_body,
        grid=grid,
        in_specs=[
            pl.BlockSpec((block_b, D), lambda i: (i, 0)),
            pl.BlockSpec((block_b, D), lambda i: (i, 0)),
            pl.BlockSpec((D, H), lambda i: (0, 0)),
            pl.BlockSpec((D, H), lambda i: (0, 0)),
            pl.BlockSpec((1, H), lambda i: (0, 0)),
            pl.BlockSpec((H, 1), lambda i: (0, 0)),
            pl.BlockSpec((1, 1), lambda i: (0, 0)),
        ],
        out_specs=pl.BlockSpec((block_b, 1), lambda i: (i, 0)),
        out_shape=jax.ShapeDtypeStruct((B, 1), jnp.float32),
    )(ue, ie, W1u, W1i, b1, W2, b2)


def kernel(user_indices, item_indices, user_table, item_table, W1, b1, W2, b2):
    D = user_table.shape[1]
    ue, ie = _gather_sc(
        user_indices.astype(jnp.int32),
        item_indices.astype(jnp.int32),
        user_table,
        item_table,
    )
    return _mlp_tc(
        ue, ie,
        W1[:D], W1[D:],
        b1.reshape(1, -1), W2, b2.reshape(1, 1),
    )
